# Initial kernel scaffold; baseline (speedup 1.0000x reference)
import jax
import jax.numpy as jnp
from jax.experimental import pallas as pl


def kernel(m, i, n, x):
    raise NotImplementedError("write your pallas kernel here")


# speedup vs baseline: 3.8298x; 3.8298x over previous
import functools

import jax
import jax.numpy as jnp
from jax import lax
from jax.experimental import pallas as pl
from jax.experimental.pallas import tpu as pltpu
from jax.experimental.pallas import tpu_sc as plsc

NC = 2
NS = 16
NW = NC * NS


def _sc_scatter(m, idx, n, zeros):
    E, D = m.shape
    C = E // NW
    T = 80
    NT = C // T
    RPW_A = ((n // NS) + 7) // 8 * 8
    RPW_B = n - (NS - 1) * RPW_A

    mesh = plsc.VectorSubcoreMesh(core_axis_name="c", subcore_axis_name="s")

    @functools.partial(
        pl.kernel,
        out_type=jax.ShapeDtypeStruct((NC * n, D), jnp.float32),
        mesh=mesh,
        scratch_types=[
            pltpu.VMEM((T,), jnp.int32),
            pltpu.VMEM((T, D), jnp.float32),
            pltpu.VMEM_SHARED((n, D), jnp.float32),
        ],
    )
    def body(m_hbm, idx_hbm, z_hbm, acc_hbm, idx_v, m_v, acc_sh):
        c = lax.axis_index("c")
        s = lax.axis_index("s")
        wid = c * NS + s

        off = pl.multiple_of(s * RPW_A, 8)

        @pl.when(s < NS - 1)
        def _():
            pltpu.sync_copy(z_hbm, acc_sh.at[pl.ds(off, RPW_A)])

        @pl.when(s == NS - 1)
        def _():
            pltpu.sync_copy(z_hbm.at[pl.ds(0, RPW_B)],
                            acc_sh.at[pl.ds((NS - 1) * RPW_A, RPW_B)])

        plsc.subcore_barrier()

        base_e = wid * C

        def tile_body(t, carry):
            e0 = pl.multiple_of(base_e + t * T, 8)
            pltpu.sync_copy(idx_hbm.at[pl.ds(e0, T)], idx_v)
            pltpu.sync_copy(m_hbm.at[pl.ds(e0, T)], m_v)
            pltpu.sync_copy(m_v, acc_sh.at[idx_v], add=True)
            return carry

        lax.fori_loop(0, NT, tile_body, 0)
        plsc.subcore_barrier()

        hoff = pl.multiple_of(c * n + s * RPW_A, 8)

        @pl.when(s < NS - 1)
        def _():
            pltpu.sync_copy(acc_sh.at[pl.ds(off, RPW_A)],
                            acc_hbm.at[pl.ds(hoff, RPW_A)])

        @pl.when(s == NS - 1)
        def _():
            pltpu.sync_copy(
                acc_sh.at[pl.ds((NS - 1) * RPW_A, RPW_B)],
                acc_hbm.at[pl.ds(pl.multiple_of(c * n + (NS - 1) * RPW_A, 8),
                                 RPW_B)],
            )

    return body(m, idx, zeros)


def _merge_body(x_ref, a0_ref, a1_ref, o_ref):
    o_ref[...] = x_ref[...] + a0_ref[...] + a1_ref[...]


def _merge(x, acc):
    N, D = x.shape
    R = 1000
    NB = N // R
    return pl.pallas_call(
        _merge_body,
        grid=(NB,),
        in_specs=[
            pl.BlockSpec((R, D), lambda b: (b, 0)),
            pl.BlockSpec((R, D), lambda b: (b, 0)),
            pl.BlockSpec((R, D), lambda b: (b + NB, 0)),
        ],
        out_specs=pl.BlockSpec((R, D), lambda b: (b, 0)),
        out_shape=jax.ShapeDtypeStruct((N, D), jnp.float32),
    )(x, acc, acc)


def kernel(m, i, n, x):
    N = x.shape[0]
    idx = jnp.asarray(i, jnp.int32)
    zeros = jnp.zeros((((N // NS) + 7) // 8 * 8, x.shape[1]), jnp.float32)
    acc = _sc_scatter(m, idx, N, zeros)
    return _merge(x, acc)


# speedup vs baseline: 8.3593x; 2.1827x over previous
import functools

import jax
import jax.numpy as jnp
from jax import lax
from jax.experimental import pallas as pl
from jax.experimental.pallas import tpu as pltpu
from jax.experimental.pallas import tpu_sc as plsc

NC = 2
NS = 16
NW = NC * NS


def _sc_scatter(m, idx, n, zeros):
    E, D = m.shape
    C = E // NW
    T = 40
    NT = C // T
    RPW_A = ((n // NS) + 7) // 8 * 8
    RPW_B = n - (NS - 1) * RPW_A

    NBUF = 5
    mesh = plsc.VectorSubcoreMesh(core_axis_name="c", subcore_axis_name="s")

    @functools.partial(
        pl.kernel,
        out_type=jax.ShapeDtypeStruct((NC * n, D), jnp.float32),
        mesh=mesh,
        scratch_types=[
            [pltpu.VMEM((T,), jnp.int32) for _ in range(NBUF)],
            [pltpu.VMEM((T, D), jnp.float32) for _ in range(NBUF)],
            [pltpu.SemaphoreType.DMA for _ in range(NBUF)],
            [pltpu.SemaphoreType.DMA for _ in range(NBUF)],
            pltpu.VMEM_SHARED((n, D), jnp.float32),
        ],
    )
    def body(m_hbm, idx_hbm, z_hbm, acc_hbm, idx_v, m_v, isem, msem, acc_sh):
        c = lax.axis_index("c")
        s = lax.axis_index("s")
        wid = c * NS + s

        base_e = wid * C

        for b in range(NBUF):
            e0 = pl.multiple_of(base_e + b * T, 8)
            pltpu.async_copy(idx_hbm.at[pl.ds(e0, T)], idx_v[b], isem[b])
            pltpu.async_copy(m_hbm.at[pl.ds(e0, T)], m_v[b], msem[b])

        off = pl.multiple_of(s * RPW_A, 8)

        @pl.when(s < NS - 1)
        def _():
            pltpu.sync_copy(z_hbm, acc_sh.at[pl.ds(off, RPW_A)])

        @pl.when(s == NS - 1)
        def _():
            pltpu.sync_copy(z_hbm.at[pl.ds(0, RPW_B)],
                            acc_sh.at[pl.ds((NS - 1) * RPW_A, RPW_B)])

        plsc.subcore_barrier()

        def group_body(g, carry):
            for b in range(NBUF):
                t = g * NBUF + b
                e0 = pl.multiple_of(base_e + t * T, 8)
                pltpu.make_async_copy(
                    idx_hbm.at[pl.ds(e0, T)], idx_v[b], isem[b]).wait()
                pltpu.make_async_copy(
                    m_hbm.at[pl.ds(e0, T)], m_v[b], msem[b]).wait()
                pltpu.sync_copy(m_v[b], acc_sh.at[idx_v[b]], add=True)

                @pl.when(t + NBUF < NT)
                def _():
                    e1 = pl.multiple_of(base_e + (t + NBUF) * T, 8)
                    pltpu.async_copy(
                        idx_hbm.at[pl.ds(e1, T)], idx_v[b], isem[b])
                    pltpu.async_copy(
                        m_hbm.at[pl.ds(e1, T)], m_v[b], msem[b])
            return carry

        lax.fori_loop(0, NT // NBUF, group_body, 0)
        plsc.subcore_barrier()

        hoff = pl.multiple_of(c * n + s * RPW_A, 8)

        @pl.when(s < NS - 1)
        def _():
            pltpu.sync_copy(acc_sh.at[pl.ds(off, RPW_A)],
                            acc_hbm.at[pl.ds(hoff, RPW_A)])

        @pl.when(s == NS - 1)
        def _():
            pltpu.sync_copy(
                acc_sh.at[pl.ds((NS - 1) * RPW_A, RPW_B)],
                acc_hbm.at[pl.ds(pl.multiple_of(c * n + (NS - 1) * RPW_A, 8),
                                 RPW_B)],
            )

    return body(m, idx, zeros)


def _merge_body(x_ref, a0_ref, a1_ref, o_ref):
    o_ref[...] = x_ref[...] + a0_ref[...] + a1_ref[...]


def _merge(x, acc):
    N, D = x.shape
    R = 1000
    NB = N // R
    return pl.pallas_call(
        _merge_body,
        grid=(NB,),
        in_specs=[
            pl.BlockSpec((R, D), lambda b: (b, 0)),
            pl.BlockSpec((R, D), lambda b: (b, 0)),
            pl.BlockSpec((R, D), lambda b: (b + NB, 0)),
        ],
        out_specs=pl.BlockSpec((R, D), lambda b: (b, 0)),
        out_shape=jax.ShapeDtypeStruct((N, D), jnp.float32),
    )(x, acc, acc)


def kernel(m, i, n, x):
    N = x.shape[0]
    idx = jnp.asarray(i, jnp.int32)
    zeros = jnp.zeros((((N // NS) + 7) // 8 * 8, x.shape[1]), jnp.float32)
    acc = _sc_scatter(m, idx, N, zeros)
    return _merge(x, acc)
